# Initial kernel scaffold; baseline (speedup 1.0000x reference)
import jax
import jax.numpy as jnp
from jax.experimental import pallas as pl


def kernel(x, embeddings):
    raise NotImplementedError("write your pallas kernel here")


# speedup vs baseline: 2.3806x; 2.3806x over previous
import jax
import jax.numpy as jnp
from jax.experimental import pallas as pl
from jax.experimental.pallas import tpu as pltpu


def _body(idx_smem, xv_ref, emb_ref, out_ref, buf_ref, sem, *, L):
    TB = out_ref.shape[0]

    def issue_rows(b, carry):
        for l in range(L):
            idx = idx_smem[b, l]
            pltpu.make_async_copy(
                emb_ref.at[idx], buf_ref.at[b, l], sem
            ).start()
        return carry

    jax.lax.fori_loop(0, TB, issue_rows, 0)

    def drain_rows(b, carry):
        for l in range(L):
            pltpu.make_async_copy(
                emb_ref.at[0], buf_ref.at[b, l], sem
            ).wait()
        return carry

    jax.lax.fori_loop(0, TB, drain_rows, 0)

    mask = (xv_ref[...] != 0).astype(jnp.float32)
    s = jnp.sum(buf_ref[...] * mask, axis=1)
    out_ref[...] = jnp.tanh(s)


def _phedvec(x, embeddings, tb, interpret=False):
    B, L = x.shape
    _, D = embeddings.shape
    nt = B // tb
    grid_spec = pltpu.PrefetchScalarGridSpec(
        num_scalar_prefetch=0,
        grid=(nt,),
        in_specs=[
            pl.BlockSpec((tb, L), lambda t: (t, 0), memory_space=pltpu.SMEM),
            pl.BlockSpec((tb, L, 1), lambda t: (t, 0, 0)),
            pl.BlockSpec(memory_space=pltpu.HBM),
        ],
        out_specs=pl.BlockSpec((tb, D), lambda t: (t, 0)),
        scratch_shapes=[
            pltpu.VMEM((tb, L, D), jnp.float32),
            pltpu.SemaphoreType.DMA,
        ],
    )
    import functools
    return pl.pallas_call(
        functools.partial(_body, L=L),
        grid_spec=grid_spec,
        out_shape=jax.ShapeDtypeStruct((B, D), jnp.float32),
        compiler_params=pltpu.CompilerParams(
            dimension_semantics=("arbitrary",),
        ),
        interpret=interpret,
    )(x, x.reshape(B, L, 1), embeddings)


def kernel(x, embeddings):
    xi = x.astype(jnp.int32)
    return _phedvec(xi, embeddings, tb=64)


# speedup vs baseline: 2.4401x; 1.0250x over previous
import functools

import jax
import jax.numpy as jnp
from jax.experimental import pallas as pl
from jax.experimental.pallas import tpu as pltpu


def _body(cur_smem, nxt_smem, xv_ref, emb_ref, dummy_ref, out_ref, buf_ref,
          sem, *, L):
    t = pl.program_id(0)
    nt = pl.num_programs(0)
    TB = out_ref.shape[0]

    def issue(idx_smem, slot):
        def issue_rows(b, carry):
            for l in range(L):
                idx = idx_smem[b, l]
                pltpu.make_async_copy(
                    emb_ref.at[idx], buf_ref.at[slot, b, l], sem.at[slot]
                ).start()
            return carry

        jax.lax.fori_loop(0, TB, issue_rows, 0)

    def drain(slot):
        def drain_rows(b, carry):
            for l in range(L):
                pltpu.make_async_copy(
                    dummy_ref.at[l], buf_ref.at[slot, b, l], sem.at[slot]
                ).wait()
            return carry

        jax.lax.fori_loop(0, TB, drain_rows, 0)

    parity = jax.lax.rem(t, 2)

    @pl.when(t == 0)
    def _():
        issue(cur_smem, 0)

    @pl.when(jnp.logical_and(t + 1 < nt, parity == 0))
    def _():
        issue(nxt_smem, 1)

    @pl.when(jnp.logical_and(t + 1 < nt, parity == 1))
    def _():
        issue(nxt_smem, 0)

    @pl.when(parity == 0)
    def _():
        drain(0)

    @pl.when(parity == 1)
    def _():
        drain(1)

    mask = (xv_ref[...] != 0).astype(jnp.float32)

    @pl.when(parity == 0)
    def _():
        out_ref[...] = jnp.tanh(jnp.sum(buf_ref[0] * mask, axis=1))

    @pl.when(parity == 1)
    def _():
        out_ref[...] = jnp.tanh(jnp.sum(buf_ref[1] * mask, axis=1))


def _phedvec(x, embeddings, tb, interpret=False):
    B, L = x.shape
    _, D = embeddings.shape
    nt = B // tb
    grid_spec = pltpu.PrefetchScalarGridSpec(
        num_scalar_prefetch=0,
        grid=(nt,),
        in_specs=[
            pl.BlockSpec((tb, L), lambda t: (t, 0), memory_space=pltpu.SMEM),
            pl.BlockSpec(
                (tb, L),
                lambda t: (jnp.minimum(t + 1, nt - 1), 0),
                memory_space=pltpu.SMEM,
            ),
            pl.BlockSpec((tb, L, 1), lambda t: (t, 0, 0)),
            pl.BlockSpec(memory_space=pltpu.HBM),
            pl.BlockSpec(memory_space=pltpu.HBM),
        ],
        out_specs=pl.BlockSpec((tb, D), lambda t: (t, 0)),
        scratch_shapes=[
            pltpu.VMEM((2, tb, L, D), jnp.float32),
            pltpu.SemaphoreType.DMA((2,)),
        ],
    )
    return pl.pallas_call(
        functools.partial(_body, L=L),
        grid_spec=grid_spec,
        out_shape=jax.ShapeDtypeStruct((B, D), jnp.float32),
        compiler_params=pltpu.CompilerParams(
            dimension_semantics=("arbitrary",),
        ),
        interpret=interpret,
    )(x, x, x.reshape(B, L, 1), embeddings,
      jnp.zeros((L, D), jnp.float32))


def kernel(x, embeddings):
    xi = x.astype(jnp.int32)
    return _phedvec(xi, embeddings, tb=64)
